# Initial kernel scaffold; baseline (speedup 1.0000x reference)
import jax
import jax.numpy as jnp
from jax.experimental import pallas as pl


def kernel(x_claim, x_user, edge_u2c, edge_c2u, W1_u2c, b1_u2c, W1_c2u, b1_c2u, W2_u2c, b2_u2c, W2_c2u, b2_c2u, ln_g, ln_b, bn_g, bn_b, lin1_W, lin1_b, lin2_W, lin2_b):
    raise NotImplementedError("write your pallas kernel here")


# speedup vs baseline: 3.2676x; 3.2676x over previous
import functools

import jax
import jax.numpy as jnp
from jax import lax
from jax.experimental import pallas as pl
from jax.experimental.pallas import tpu as pltpu
from jax.experimental.pallas import tpu_sc as plsc

N = 10000
H = 128
E = 320000
NC = 2
NS = 16
B = 80
C = E // (NC * NS * B)
NP = 10240
RPT = NP // NS
DW = 8
HALF = NP // 2
ACCR = HALF + 8
RPC = HALF // NS

_mesh = plsc.VectorSubcoreMesh(core_axis_name="c", subcore_axis_name="s")


C2 = E // (NS * B)


def _sc_degree_body(idx_hbm, out_hbm, hist, idx_v):
    c = lax.axis_index("c")
    s = lax.axis_index("s")

    def per_array(t, _):
        def zfill(i, __):
            hist[pl.ds(i * 16, 16)] = jnp.zeros((16,), jnp.float32)
            return __
        lax.fori_loop(0, NP // 16, zfill, 0)

        pltpu.sync_copy(idx_hbm.at[2 * c + t].at[s], idx_v)
        ones16 = jnp.ones((16,), jnp.float32)

        def count(i, __):
            j = i // (B // 16)
            k = i % (B // 16)
            v = idx_v[j, pl.ds(k * 16, 16)]
            plsc.addupdate_scatter(hist, [v], ones16)
            return __
        lax.fori_loop(0, C2 * (B // 16), count, 0)
        pltpu.sync_copy(hist, out_hbm.at[2 * c + t].at[s])
        return _
    lax.fori_loop(0, 2, per_array, 0)


@functools.partial(
    pl.kernel,
    out_type=jax.ShapeDtypeStruct((4, NS, NP), jnp.float32),
    mesh=_mesh,
    scratch_types=[
        pltpu.VMEM((NP,), jnp.float32),
        pltpu.VMEM((C2, B), jnp.int32),
    ],
    compiler_params=pltpu.CompilerParams(needs_layout_passes=False),
)
def _sc_degrees(idx_hbm, out_hbm, hist, idx_v):
    _sc_degree_body(idx_hbm, out_hbm, hist, idx_v)


def _degscale_body(dh_ref, i_ref, o_ref):
    d = jnp.sum(dh_ref[...], axis=1)
    sc = lax.rsqrt(jnp.maximum(d, 1.0))
    eye = i_ref[...]
    for a in range(4):
        col = lax.dot_general(eye, sc[a:a + 1, :], (((1,), (1,)), ((), ())),
                              preferred_element_type=jnp.float32)
        o_ref[a] = jnp.broadcast_to(col, (128, DW))


def _tc_degscale(dh, eye):
    return pl.pallas_call(
        _degscale_body,
        grid=(NP // 128,),
        in_specs=[
            pl.BlockSpec((4, NS, 128), lambda i: (0, 0, i)),
            pl.BlockSpec((128, 128), lambda i: (0, 0)),
        ],
        out_specs=pl.BlockSpec((4, 128, DW), lambda i: (0, i, 0)),
        out_shape=jax.ShapeDtypeStruct((4, NP, DW), jnp.float32),
    )(dh, eye)


ZR = 8


def _sc_conv_body(table_hbm, src_hbm, dst_hbm, out_hbm, idx_s, idx_r,
                  rows, zb, sem, acc):
    c = lax.axis_index("c")
    s = lax.axis_index("s")
    base = s * RPC

    def zr(r, _):
        def zc(k, __):
            zb[r, pl.ds(k * 16, 16)] = jnp.zeros((16,), jnp.float32)
            return __
        return lax.fori_loop(0, H // 16, zc, _)
    lax.fori_loop(0, ZR, zr, 0)

    pltpu.sync_copy(src_hbm.at[c].at[s], idx_s)

    def one_pass(p, _):
        def zcp(k, __):
            pltpu.sync_copy(zb, acc.at[pl.ds(base + k * ZR, ZR)])
            return __
        lax.fori_loop(0, RPC // ZR, zcp, 0)

        @pl.when(s == 0)
        def _zt():
            pltpu.sync_copy(zb.at[pl.ds(0, 8)], acc.at[pl.ds(HALF, 8)])

        lo = p * HALF
        pltpu.sync_copy(dst_hbm.at[c].at[s], idx_r)

        def remap(i, __):
            j = i // (B // 16)
            k = i % (B // 16)
            d = idx_r[j, pl.ds(k * 16, 16)]
            loc = d - lo
            ok = (loc >= 0) & (loc < HALF)
            idx_r[j, pl.ds(k * 16, 16)] = jnp.where(ok, loc, HALF)
            return __
        lax.fori_loop(0, C * (B // 16), remap, 0)
        plsc.subcore_barrier()

        def chunk(j, __):
            pltpu.async_copy(table_hbm.at[idx_s.at[j]], rows, sem).wait()
            pltpu.sync_copy(rows, acc.at[idx_r.at[j]], add=True)
            return __
        lax.fori_loop(0, C, chunk, 0)
        plsc.subcore_barrier()

        pltpu.sync_copy(acc.at[pl.ds(base, RPC)],
                        out_hbm.at[c].at[pl.ds(lo + base, RPC)])
        plsc.subcore_barrier()
        return _
    lax.fori_loop(0, 2, one_pass, 0)


@functools.partial(
    pl.kernel,
    out_type=jax.ShapeDtypeStruct((NC, NP, H), jnp.float32),
    mesh=_mesh,
    scratch_types=[
        pltpu.VMEM((C, B), jnp.int32),
        pltpu.VMEM((C, B), jnp.int32),
        pltpu.VMEM((B, H), jnp.float32),
        pltpu.VMEM((ZR, H), jnp.float32),
        pltpu.SemaphoreType.DMA,
        pltpu.VMEM_SHARED((ACCR, H), jnp.float32),
    ],
)
def _sc_conv(table_hbm, src_hbm, dst_hbm, out_hbm, idx_s, idx_r, rows,
             zb, sem, acc):
    _sc_conv_body(table_hbm, src_hbm, dst_hbm, out_hbm, idx_s, idx_r,
                  rows, zb, sem, acc)


_INV_SQRT2 = 0.7071067811865476


def _gelu(x):
    return 0.5 * x * (1.0 + lax.erf(x * _INV_SQRT2))


def _ln(x, g, b):
    mu = jnp.mean(x, axis=-1, keepdims=True)
    d = x - mu
    var = jnp.mean(d * d, axis=-1, keepdims=True)
    return d * lax.rsqrt(var + 1e-5) * g + b


def _scale_col(d_ref):
    return d_ref[:, :1]


NB = 2000


def _prep_body(x_ref, d_ref, o_ref):
    o_ref[...] = x_ref[...] * _scale_col(d_ref)


def _tc_prep(x, d_so):
    return pl.pallas_call(
        _prep_body,
        grid=(N // NB,),
        in_specs=[
            pl.BlockSpec((NB, H), lambda i: (i, 0)),
            pl.BlockSpec((NB, DW), lambda i: (i, 0)),
        ],
        out_specs=pl.BlockSpec((NB, H), lambda i: (i, 0)),
        out_shape=jax.ShapeDtypeStruct((N, H), jnp.float32),
    )(x, d_so)


def _mid_body(p_ref, dsi_ref, dso_ref, w_ref, b_ref, g_ref, be_ref, o_ref):
    agg = (p_ref[0] + p_ref[1]) * _scale_col(dsi_ref)
    z = jnp.dot(agg, w_ref[...], preferred_element_type=jnp.float32)
    y = _ln(_gelu(z + b_ref[...]), g_ref[...], be_ref[...])
    o_ref[...] = y * _scale_col(dso_ref)


def _tc_mid(p, d_si, d_so, w, b, g, be):
    vec = pl.BlockSpec((1, H), lambda i: (0, 0))
    deg = pl.BlockSpec((NB, DW), lambda i: (i, 0))
    return pl.pallas_call(
        _mid_body,
        grid=(N // NB,),
        in_specs=[
            pl.BlockSpec((NC, NB, H), lambda i: (0, i, 0)),
            deg,
            deg,
            pl.BlockSpec((H, H), lambda i: (0, 0)),
            vec, vec, vec,
        ],
        out_specs=pl.BlockSpec((NB, H), lambda i: (i, 0)),
        out_shape=jax.ShapeDtypeStruct((N, H), jnp.float32),
    )(p, d_si, d_so, w, b, g, be)


def _head_body(p_ref, dsi_ref, w2_ref, b2_ref, g_ref, be_ref, bg_ref, bb_ref,
               w3_ref, b3_ref, w4_ref, b4_ref, o_ref):
    agg = (p_ref[0] + p_ref[1]) * _scale_col(dsi_ref)
    z = jnp.dot(agg, w2_ref[...], preferred_element_type=jnp.float32)
    c2 = _ln(_gelu(z + b2_ref[...]), g_ref[...], be_ref[...])
    mu = jnp.mean(c2, axis=0, keepdims=True)
    d = c2 - mu
    var = jnp.mean(d * d, axis=0, keepdims=True)
    h = d * lax.rsqrt(var + 1e-5) * bg_ref[...] + bb_ref[...]
    h = _gelu(jnp.dot(h, w3_ref[...], preferred_element_type=jnp.float32)
              + b3_ref[...])
    o_ref[...] = (jnp.dot(h, w4_ref[...], preferred_element_type=jnp.float32)
                  + b4_ref[...])


def _tc_head(p, d_si, w2, b2, g, be, bg, bb, w3, b3, w4, b4):
    return pl.pallas_call(
        _head_body,
        out_shape=jax.ShapeDtypeStruct((N, H), jnp.float32),
    )(p, d_si, w2, b2, g, be, bg, bb, w3, b3, w4, b4)


def kernel(x_claim, x_user, edge_u2c, edge_c2u, W1_u2c, b1_u2c, W1_c2u,
           b1_c2u, W2_u2c, b2_u2c, W2_c2u, b2_c2u, ln_g, ln_b, bn_g, bn_b,
           lin1_W, lin1_b, lin2_W, lin2_b):
    e_u2c = edge_u2c.astype(jnp.int32)
    e_c2u = edge_c2u.astype(jnp.int32)

    idx4 = jnp.stack([e_c2u[0], e_c2u[1], e_u2c[0], e_u2c[1]])
    idx4 = idx4.reshape(4, NS, C2, B)
    dh = _sc_degrees(idx4)
    scales = _tc_degscale(dh, jnp.eye(128, dtype=jnp.float32))
    d_so_c = scales[0, :N]
    d_si_u = scales[1, :N]
    d_so_u = scales[2, :N]
    d_si_c = scales[3, :N]

    row = lambda v: v.reshape(1, H)

    h1 = _tc_prep(x_claim, d_so_c)
    p1 = _sc_conv(h1, e_c2u[0].reshape(NC, NS, C, B),
                  e_c2u[1].reshape(NC, NS, C, B))[:, :N]
    h2 = _tc_mid(p1, d_si_u, d_so_u, W1_c2u, row(b1_c2u), row(ln_g),
                 row(ln_b))

    p2 = _sc_conv(h2, e_u2c[0].reshape(NC, NS, C, B),
                  e_u2c[1].reshape(NC, NS, C, B))[:, :N]
    w4 = jnp.pad(lin2_W, ((0, 0), (0, H - 1)))
    b4 = jnp.pad(lin2_b, (0, H - 1)).reshape(1, H)
    out = _tc_head(p2, d_si_c, W2_u2c, row(b2_u2c), row(ln_g), row(ln_b),
                   row(bn_g), row(bn_b), lin1_W, row(lin1_b), w4, b4)
    return out[:, :1]


# speedup vs baseline: 4.0913x; 1.2521x over previous
import functools

import jax
import jax.numpy as jnp
from jax import lax
from jax.experimental import pallas as pl
from jax.experimental.pallas import tpu as pltpu
from jax.experimental.pallas import tpu_sc as plsc

N = 10000
H = 128
E = 320000
NC = 2
NS = 16
B = 80
C = E // (NC * NS * B)
NP = 10240
RPT = NP // NS
DW = 8
HALF = NP // 2
ACCR = HALF + 8
RPC = HALF // NS

_mesh = plsc.VectorSubcoreMesh(core_axis_name="c", subcore_axis_name="s")


C2 = E // (NS * B)


def _sc_degree_body(idx_hbm, out_hbm, hist, idx_v):
    c = lax.axis_index("c")
    s = lax.axis_index("s")

    def per_array(t, _):
        def zfill(i, __):
            hist[pl.ds(i * 16, 16)] = jnp.zeros((16,), jnp.float32)
            return __
        lax.fori_loop(0, NP // 16, zfill, 0)

        pltpu.sync_copy(idx_hbm.at[2 * c + t].at[s], idx_v)
        ones16 = jnp.ones((16,), jnp.float32)

        def count(i, __):
            j = i // (B // 16)
            k = i % (B // 16)
            v = idx_v[j, pl.ds(k * 16, 16)]
            plsc.addupdate_scatter(hist, [v], ones16)
            return __
        lax.fori_loop(0, C2 * (B // 16), count, 0)
        pltpu.sync_copy(hist, out_hbm.at[2 * c + t].at[s])
        return _
    lax.fori_loop(0, 2, per_array, 0)


@functools.partial(
    pl.kernel,
    out_type=jax.ShapeDtypeStruct((4, NS, NP), jnp.float32),
    mesh=_mesh,
    scratch_types=[
        pltpu.VMEM((NP,), jnp.float32),
        pltpu.VMEM((C2, B), jnp.int32),
    ],
    compiler_params=pltpu.CompilerParams(needs_layout_passes=False),
)
def _sc_degrees(idx_hbm, out_hbm, hist, idx_v):
    _sc_degree_body(idx_hbm, out_hbm, hist, idx_v)


def _degscale_body(dh_ref, i_ref, o_ref):
    d = jnp.sum(dh_ref[...], axis=1)
    sc = lax.rsqrt(jnp.maximum(d, 1.0))
    eye = i_ref[...]
    for a in range(4):
        col = lax.dot_general(eye, sc[a:a + 1, :], (((1,), (1,)), ((), ())),
                              preferred_element_type=jnp.float32)
        o_ref[a] = jnp.broadcast_to(col, (128, DW))


def _tc_degscale(dh, eye):
    return pl.pallas_call(
        _degscale_body,
        grid=(NP // 128,),
        in_specs=[
            pl.BlockSpec((4, NS, 128), lambda i: (0, 0, i)),
            pl.BlockSpec((128, 128), lambda i: (0, 0)),
        ],
        out_specs=pl.BlockSpec((4, 128, DW), lambda i: (0, i, 0)),
        out_shape=jax.ShapeDtypeStruct((4, NP, DW), jnp.float32),
    )(dh, eye)


ZR = 8


def _sc_conv_body(table_hbm, src_hbm, dst_hbm, out_hbm, idx_s, idx_r,
                  rows, zb, sem, acc):
    c = lax.axis_index("c")
    s = lax.axis_index("s")
    base = s * RPC

    def zr(r, _):
        def zc(k, __):
            zb[r, pl.ds(k * 16, 16)] = jnp.zeros((16,), jnp.float32)
            return __
        return lax.fori_loop(0, H // 16, zc, _)
    lax.fori_loop(0, ZR, zr, 0)

    pltpu.sync_copy(src_hbm.at[c].at[s], idx_s)

    def one_pass(p, _):
        def zcp(k, __):
            pltpu.sync_copy(zb, acc.at[pl.ds(base + k * ZR, ZR)])
            return __
        lax.fori_loop(0, RPC // ZR, zcp, 0)

        @pl.when(s == 0)
        def _zt():
            pltpu.sync_copy(zb.at[pl.ds(0, 8)], acc.at[pl.ds(HALF, 8)])

        lo = p * HALF
        pltpu.sync_copy(dst_hbm.at[c].at[s], idx_r)

        def remap(i, __):
            j = i // (B // 16)
            k = i % (B // 16)
            d = idx_r[j, pl.ds(k * 16, 16)]
            loc = d - lo
            ok = (loc >= 0) & (loc < HALF)
            idx_r[j, pl.ds(k * 16, 16)] = jnp.where(ok, loc, HALF)
            return __
        lax.fori_loop(0, C * (B // 16), remap, 0)
        plsc.subcore_barrier()

        pltpu.async_copy(table_hbm.at[idx_s.at[0]], rows.at[0], sem)

        def chunk(j, __):
            slot = lax.rem(j, 2)
            pltpu.make_async_copy(table_hbm.at[idx_s.at[j]], rows.at[slot],
                                  sem).wait()

            @pl.when(j + 1 < C)
            def _nx():
                pltpu.async_copy(table_hbm.at[idx_s.at[j + 1]],
                                 rows.at[lax.rem(j + 1, 2)], sem)
            pltpu.sync_copy(rows.at[slot], acc.at[idx_r.at[j]], add=True)
            return __
        lax.fori_loop(0, C, chunk, 0)
        plsc.subcore_barrier()

        pltpu.sync_copy(acc.at[pl.ds(base, RPC)],
                        out_hbm.at[c].at[pl.ds(lo + base, RPC)])
        plsc.subcore_barrier()
        return _
    lax.fori_loop(0, 2, one_pass, 0)


@functools.partial(
    pl.kernel,
    out_type=jax.ShapeDtypeStruct((NC, NP, H), jnp.float32),
    mesh=_mesh,
    scratch_types=[
        pltpu.VMEM((C, B), jnp.int32),
        pltpu.VMEM((C, B), jnp.int32),
        pltpu.VMEM((2, B, H), jnp.float32),
        pltpu.VMEM((ZR, H), jnp.float32),
        pltpu.SemaphoreType.DMA,
        pltpu.VMEM_SHARED((ACCR, H), jnp.float32),
    ],
)
def _sc_conv(table_hbm, src_hbm, dst_hbm, out_hbm, idx_s, idx_r, rows,
             zb, sem, acc):
    _sc_conv_body(table_hbm, src_hbm, dst_hbm, out_hbm, idx_s, idx_r,
                  rows, zb, sem, acc)


_INV_SQRT2 = 0.7071067811865476


def _gelu(x):
    return 0.5 * x * (1.0 + lax.erf(x * _INV_SQRT2))


def _ln(x, g, b):
    mu = jnp.mean(x, axis=-1, keepdims=True)
    d = x - mu
    var = jnp.mean(d * d, axis=-1, keepdims=True)
    return d * lax.rsqrt(var + 1e-5) * g + b


def _scale_col(d_ref):
    return d_ref[:, :1]


NB = 2000


def _prep_body(x_ref, d_ref, o_ref):
    o_ref[...] = x_ref[...] * _scale_col(d_ref)


def _tc_prep(x, d_so):
    return pl.pallas_call(
        _prep_body,
        grid=(N // NB,),
        in_specs=[
            pl.BlockSpec((NB, H), lambda i: (i, 0)),
            pl.BlockSpec((NB, DW), lambda i: (i, 0)),
        ],
        out_specs=pl.BlockSpec((NB, H), lambda i: (i, 0)),
        out_shape=jax.ShapeDtypeStruct((N, H), jnp.float32),
    )(x, d_so)


def _mid_body(p_ref, dsi_ref, dso_ref, w_ref, b_ref, g_ref, be_ref, o_ref):
    agg = (p_ref[0] + p_ref[1]) * _scale_col(dsi_ref)
    z = jnp.dot(agg, w_ref[...], preferred_element_type=jnp.float32)
    y = _ln(_gelu(z + b_ref[...]), g_ref[...], be_ref[...])
    o_ref[...] = y * _scale_col(dso_ref)


def _tc_mid(p, d_si, d_so, w, b, g, be):
    vec = pl.BlockSpec((1, H), lambda i: (0, 0))
    deg = pl.BlockSpec((NB, DW), lambda i: (i, 0))
    return pl.pallas_call(
        _mid_body,
        grid=(N // NB,),
        in_specs=[
            pl.BlockSpec((NC, NB, H), lambda i: (0, i, 0)),
            deg,
            deg,
            pl.BlockSpec((H, H), lambda i: (0, 0)),
            vec, vec, vec,
        ],
        out_specs=pl.BlockSpec((NB, H), lambda i: (i, 0)),
        out_shape=jax.ShapeDtypeStruct((N, H), jnp.float32),
    )(p, d_si, d_so, w, b, g, be)


def _head_body(p_ref, dsi_ref, w2_ref, b2_ref, g_ref, be_ref, bg_ref, bb_ref,
               w3_ref, b3_ref, w4_ref, b4_ref, o_ref):
    agg = (p_ref[0] + p_ref[1]) * _scale_col(dsi_ref)
    z = jnp.dot(agg, w2_ref[...], preferred_element_type=jnp.float32)
    c2 = _ln(_gelu(z + b2_ref[...]), g_ref[...], be_ref[...])
    mu = jnp.mean(c2, axis=0, keepdims=True)
    d = c2 - mu
    var = jnp.mean(d * d, axis=0, keepdims=True)
    h = d * lax.rsqrt(var + 1e-5) * bg_ref[...] + bb_ref[...]
    h = _gelu(jnp.dot(h, w3_ref[...], preferred_element_type=jnp.float32)
              + b3_ref[...])
    o_ref[...] = (jnp.dot(h, w4_ref[...], preferred_element_type=jnp.float32)
                  + b4_ref[...])


def _tc_head(p, d_si, w2, b2, g, be, bg, bb, w3, b3, w4, b4):
    return pl.pallas_call(
        _head_body,
        out_shape=jax.ShapeDtypeStruct((N, H), jnp.float32),
    )(p, d_si, w2, b2, g, be, bg, bb, w3, b3, w4, b4)


def kernel(x_claim, x_user, edge_u2c, edge_c2u, W1_u2c, b1_u2c, W1_c2u,
           b1_c2u, W2_u2c, b2_u2c, W2_c2u, b2_c2u, ln_g, ln_b, bn_g, bn_b,
           lin1_W, lin1_b, lin2_W, lin2_b):
    e_u2c = edge_u2c.astype(jnp.int32)
    e_c2u = edge_c2u.astype(jnp.int32)

    idx4 = jnp.stack([e_c2u[0], e_c2u[1], e_u2c[0], e_u2c[1]])
    idx4 = idx4.reshape(4, NS, C2, B)
    dh = _sc_degrees(idx4)
    scales = _tc_degscale(dh, jnp.eye(128, dtype=jnp.float32))
    d_so_c = scales[0, :N]
    d_si_u = scales[1, :N]
    d_so_u = scales[2, :N]
    d_si_c = scales[3, :N]

    row = lambda v: v.reshape(1, H)

    h1 = _tc_prep(x_claim, d_so_c)
    p1 = _sc_conv(h1, e_c2u[0].reshape(NC, NS, C, B),
                  e_c2u[1].reshape(NC, NS, C, B))[:, :N]
    h2 = _tc_mid(p1, d_si_u, d_so_u, W1_c2u, row(b1_c2u), row(ln_g),
                 row(ln_b))

    p2 = _sc_conv(h2, e_u2c[0].reshape(NC, NS, C, B),
                  e_u2c[1].reshape(NC, NS, C, B))[:, :N]
    w4 = jnp.pad(lin2_W, ((0, 0), (0, H - 1)))
    b4 = jnp.pad(lin2_b, (0, H - 1)).reshape(1, H)
    out = _tc_head(p2, d_si_c, W2_u2c, row(b2_u2c), row(ln_g), row(ln_b),
                   row(bn_g), row(bn_b), lin1_W, row(lin1_b), w4, b4)
    return out[:, :1]


# speedup vs baseline: 4.2257x; 1.0329x over previous
import functools

import jax
import jax.numpy as jnp
from jax import lax
from jax.experimental import pallas as pl
from jax.experimental.pallas import tpu as pltpu
from jax.experimental.pallas import tpu_sc as plsc

N = 10000
H = 128
E = 320000
NC = 2
NS = 16
B = 80
C = E // (NC * NS * B)
NP = 10240
RPT = NP // NS
DW = 8
HALF = NP // 2
ACCR = HALF + 8
RPC = HALF // NS

_mesh = plsc.VectorSubcoreMesh(core_axis_name="c", subcore_axis_name="s")


C2 = E // (NS * B)


def _sc_degree_body(idx_hbm, out_hbm, hist, idx_v):
    c = lax.axis_index("c")
    s = lax.axis_index("s")

    def per_array(t, _):
        def zfill(i, __):
            hist[pl.ds(i * 16, 16)] = jnp.zeros((16,), jnp.float32)
            return __
        lax.fori_loop(0, NP // 16, zfill, 0)

        pltpu.sync_copy(idx_hbm.at[2 * c + t].at[s], idx_v)
        ones16 = jnp.ones((16,), jnp.float32)

        def count(i, __):
            j = i // (B // 16)
            k = i % (B // 16)
            v = idx_v[j, pl.ds(k * 16, 16)]
            plsc.addupdate_scatter(hist, [v], ones16)
            return __
        lax.fori_loop(0, C2 * (B // 16), count, 0)
        pltpu.sync_copy(hist, out_hbm.at[2 * c + t].at[s])
        return _
    lax.fori_loop(0, 2, per_array, 0)


@functools.partial(
    pl.kernel,
    out_type=jax.ShapeDtypeStruct((4, NS, NP), jnp.float32),
    mesh=_mesh,
    scratch_types=[
        pltpu.VMEM((NP,), jnp.float32),
        pltpu.VMEM((C2, B), jnp.int32),
    ],
    compiler_params=pltpu.CompilerParams(needs_layout_passes=False),
)
def _sc_degrees(idx_hbm, out_hbm, hist, idx_v):
    _sc_degree_body(idx_hbm, out_hbm, hist, idx_v)


def _degscale_body(dh_ref, i_ref, o_ref):
    d = jnp.sum(dh_ref[...], axis=1)
    sc = lax.rsqrt(jnp.maximum(d, 1.0))
    eye = i_ref[...]
    for a in range(4):
        col = lax.dot_general(eye, sc[a:a + 1, :], (((1,), (1,)), ((), ())),
                              preferred_element_type=jnp.float32)
        o_ref[a] = jnp.broadcast_to(col, (128, DW))


def _tc_degscale(dh, eye):
    return pl.pallas_call(
        _degscale_body,
        grid=(NP // 128,),
        in_specs=[
            pl.BlockSpec((4, NS, 128), lambda i: (0, 0, i)),
            pl.BlockSpec((128, 128), lambda i: (0, 0)),
        ],
        out_specs=pl.BlockSpec((4, 128, DW), lambda i: (0, i, 0)),
        out_shape=jax.ShapeDtypeStruct((4, NP, DW), jnp.float32),
    )(dh, eye)


ZR = 8


EPT = E // (NC * NS)
BL = 128
LB = 10496


def _sc_conv_body(table_hbm, src_hbm, dst_hbm, out_hbm, idx_s, idx_d,
                  src_c, dst_c, rows, zb, sem, acc):
    c = lax.axis_index("c")
    s = lax.axis_index("s")
    base = s * RPC

    def zr(r, _):
        def zc(k, __):
            zb[r, pl.ds(k * 16, 16)] = jnp.zeros((16,), jnp.float32)
            return __
        return lax.fori_loop(0, H // 16, zc, _)
    lax.fori_loop(0, ZR, zr, 0)

    pltpu.sync_copy(src_hbm.at[c].at[s], idx_s)
    pltpu.sync_copy(dst_hbm.at[c].at[s], idx_d)

    def pfill(i, _):
        src_c[pl.ds(i * 16, 16)] = jnp.zeros((16,), jnp.int32)
        dst_c[pl.ds(i * 16, 16)] = jnp.full((16,), HALF, jnp.int32)
        return _
    lax.fori_loop(0, LB // 16, pfill, 0)

    def cnt(i, off):
        j = i // (B // 16)
        k = i % (B // 16)
        dv = idx_d[j, pl.ds(k * 16, 16)]
        return off + jnp.sum((dv < HALF).astype(jnp.int32))
    n_lo = lax.fori_loop(0, C * (B // 16), cnt, 0)
    hb = pl.multiple_of(((n_lo + 16 + BL - 1) // BL) * BL, BL)

    def compact(i, offs):
        off_lo, off_hi = offs
        j = i // (B // 16)
        k = i % (B // 16)
        sv = idx_s[j, pl.ds(k * 16, 16)]
        dv = idx_d[j, pl.ds(k * 16, 16)]
        m_lo = dv < HALF
        plsc.store_compressed(src_c.at[pl.ds(off_lo, 16)], sv, mask=m_lo)
        plsc.store_compressed(dst_c.at[pl.ds(off_lo, 16)], dv, mask=m_lo)
        m_hi = jnp.logical_not(m_lo)
        plsc.store_compressed(src_c.at[pl.ds(off_hi, 16)], sv, mask=m_hi)
        plsc.store_compressed(dst_c.at[pl.ds(off_hi, 16)], dv - HALF,
                              mask=m_hi)
        n = jnp.sum(m_lo.astype(jnp.int32))
        return off_lo + n, off_hi + (16 - n)
    off_lo_f, off_hi_f = lax.fori_loop(0, C * (B // 16), compact, (0, hb))
    n_hi = off_hi_f - hb

    @pl.when(off_lo_f < hb)
    def _pt1():
        src_c[pl.ds(off_lo_f, 16)] = jnp.zeros((16,), jnp.int32)
        dst_c[pl.ds(off_lo_f, 16)] = jnp.full((16,), HALF, jnp.int32)

    src_c[pl.ds(off_hi_f, 16)] = jnp.zeros((16,), jnp.int32)
    dst_c[pl.ds(off_hi_f, 16)] = jnp.full((16,), HALF, jnp.int32)

    for p, nb in ((0, jnp.int32(0)), (1, hb)):
        n_p = jnp.where(p == 0, n_lo, n_hi)
        def zcp(k, __):
            pltpu.sync_copy(zb, acc.at[pl.ds(base + k * ZR, ZR)])
            return __
        lax.fori_loop(0, RPC // ZR, zcp, 0)

        @pl.when(s == 0)
        def _zt():
            pltpu.sync_copy(zb.at[pl.ds(0, 8)], acc.at[pl.ds(HALF, 8)])
        plsc.subcore_barrier()

        n_chunks = (n_p + BL - 1) // BL

        def gwin(j):
            return pl.multiple_of(nb + j * BL, BL)

        @pl.when(n_chunks > 0)
        def _pro():
            pltpu.async_copy(table_hbm.at[src_c.at[pl.ds(gwin(0), BL)]],
                             rows.at[0], sem)

        def chunk(j, __):
            slot = lax.rem(j, 2)
            pltpu.make_async_copy(
                table_hbm.at[src_c.at[pl.ds(gwin(j), BL)]],
                rows.at[slot], sem).wait()

            @pl.when(j + 1 < n_chunks)
            def _nx():
                pltpu.async_copy(
                    table_hbm.at[src_c.at[pl.ds(gwin(j + 1), BL)]],
                    rows.at[lax.rem(j + 1, 2)], sem)
            pltpu.sync_copy(rows.at[slot],
                            acc.at[dst_c.at[pl.ds(gwin(j), BL)]],
                            add=True)
            return __
        lax.fori_loop(0, n_chunks, chunk, 0)
        plsc.subcore_barrier()

        pltpu.sync_copy(acc.at[pl.ds(base, RPC)],
                        out_hbm.at[c].at[pl.ds(p * HALF + base, RPC)])
        plsc.subcore_barrier()


@functools.partial(
    pl.kernel,
    out_type=jax.ShapeDtypeStruct((NC, NP, H), jnp.float32),
    mesh=_mesh,
    scratch_types=[
        pltpu.VMEM((C, B), jnp.int32),
        pltpu.VMEM((C, B), jnp.int32),
        pltpu.VMEM((LB,), jnp.int32),
        pltpu.VMEM((LB,), jnp.int32),
        pltpu.VMEM((2, BL, H), jnp.float32),
        pltpu.VMEM((ZR, H), jnp.float32),
        pltpu.SemaphoreType.DMA,
        pltpu.VMEM_SHARED((ACCR, H), jnp.float32),
    ],
    compiler_params=pltpu.CompilerParams(needs_layout_passes=False),
)
def _sc_conv(table_hbm, src_hbm, dst_hbm, out_hbm, idx_s, idx_d, src_c,
             dst_c, rows, zb, sem, acc):
    _sc_conv_body(table_hbm, src_hbm, dst_hbm, out_hbm, idx_s, idx_d,
                  src_c, dst_c, rows, zb, sem, acc)


_INV_SQRT2 = 0.7071067811865476


def _gelu(x):
    return 0.5 * x * (1.0 + lax.erf(x * _INV_SQRT2))


def _ln(x, g, b):
    mu = jnp.mean(x, axis=-1, keepdims=True)
    d = x - mu
    var = jnp.mean(d * d, axis=-1, keepdims=True)
    return d * lax.rsqrt(var + 1e-5) * g + b


def _scale_col(d_ref):
    return d_ref[:, :1]


NB = 2000


def _prep_body(x_ref, d_ref, o_ref):
    o_ref[...] = x_ref[...] * _scale_col(d_ref)


def _tc_prep(x, d_so):
    return pl.pallas_call(
        _prep_body,
        grid=(N // NB,),
        in_specs=[
            pl.BlockSpec((NB, H), lambda i: (i, 0)),
            pl.BlockSpec((NB, DW), lambda i: (i, 0)),
        ],
        out_specs=pl.BlockSpec((NB, H), lambda i: (i, 0)),
        out_shape=jax.ShapeDtypeStruct((N, H), jnp.float32),
    )(x, d_so)


def _mid_body(p_ref, dsi_ref, dso_ref, w_ref, b_ref, g_ref, be_ref, o_ref):
    agg = (p_ref[0] + p_ref[1]) * _scale_col(dsi_ref)
    z = jnp.dot(agg, w_ref[...], preferred_element_type=jnp.float32)
    y = _ln(_gelu(z + b_ref[...]), g_ref[...], be_ref[...])
    o_ref[...] = y * _scale_col(dso_ref)


def _tc_mid(p, d_si, d_so, w, b, g, be):
    vec = pl.BlockSpec((1, H), lambda i: (0, 0))
    deg = pl.BlockSpec((NB, DW), lambda i: (i, 0))
    return pl.pallas_call(
        _mid_body,
        grid=(N // NB,),
        in_specs=[
            pl.BlockSpec((NC, NB, H), lambda i: (0, i, 0)),
            deg,
            deg,
            pl.BlockSpec((H, H), lambda i: (0, 0)),
            vec, vec, vec,
        ],
        out_specs=pl.BlockSpec((NB, H), lambda i: (i, 0)),
        out_shape=jax.ShapeDtypeStruct((N, H), jnp.float32),
    )(p, d_si, d_so, w, b, g, be)


def _head_body(p_ref, dsi_ref, w2_ref, b2_ref, g_ref, be_ref, bg_ref, bb_ref,
               w3_ref, b3_ref, w4_ref, b4_ref, o_ref):
    agg = (p_ref[0] + p_ref[1]) * _scale_col(dsi_ref)
    z = jnp.dot(agg, w2_ref[...], preferred_element_type=jnp.float32)
    c2 = _ln(_gelu(z + b2_ref[...]), g_ref[...], be_ref[...])
    mu = jnp.mean(c2, axis=0, keepdims=True)
    d = c2 - mu
    var = jnp.mean(d * d, axis=0, keepdims=True)
    h = d * lax.rsqrt(var + 1e-5) * bg_ref[...] + bb_ref[...]
    h = _gelu(jnp.dot(h, w3_ref[...], preferred_element_type=jnp.float32)
              + b3_ref[...])
    o_ref[...] = (jnp.dot(h, w4_ref[...], preferred_element_type=jnp.float32)
                  + b4_ref[...])


def _tc_head(p, d_si, w2, b2, g, be, bg, bb, w3, b3, w4, b4):
    return pl.pallas_call(
        _head_body,
        out_shape=jax.ShapeDtypeStruct((N, H), jnp.float32),
    )(p, d_si, w2, b2, g, be, bg, bb, w3, b3, w4, b4)


def kernel(x_claim, x_user, edge_u2c, edge_c2u, W1_u2c, b1_u2c, W1_c2u,
           b1_c2u, W2_u2c, b2_u2c, W2_c2u, b2_c2u, ln_g, ln_b, bn_g, bn_b,
           lin1_W, lin1_b, lin2_W, lin2_b):
    e_u2c = edge_u2c.astype(jnp.int32)
    e_c2u = edge_c2u.astype(jnp.int32)

    idx4 = jnp.stack([e_c2u[0], e_c2u[1], e_u2c[0], e_u2c[1]])
    idx4 = idx4.reshape(4, NS, C2, B)
    dh = _sc_degrees(idx4)
    scales = _tc_degscale(dh, jnp.eye(128, dtype=jnp.float32))
    d_so_c = scales[0, :N]
    d_si_u = scales[1, :N]
    d_so_u = scales[2, :N]
    d_si_c = scales[3, :N]

    row = lambda v: v.reshape(1, H)

    h1 = _tc_prep(x_claim, d_so_c)
    p1 = _sc_conv(h1, e_c2u[0].reshape(NC, NS, C, B),
                  e_c2u[1].reshape(NC, NS, C, B))[:, :N]
    h2 = _tc_mid(p1, d_si_u, d_so_u, W1_c2u, row(b1_c2u), row(ln_g),
                 row(ln_b))

    p2 = _sc_conv(h2, e_u2c[0].reshape(NC, NS, C, B),
                  e_u2c[1].reshape(NC, NS, C, B))[:, :N]
    w4 = jnp.pad(lin2_W, ((0, 0), (0, H - 1)))
    b4 = jnp.pad(lin2_b, (0, H - 1)).reshape(1, H)
    out = _tc_head(p2, d_si_c, W2_u2c, row(b2_u2c), row(ln_g), row(ln_b),
                   row(bn_g), row(bn_b), lin1_W, row(lin1_b), w4, b4)
    return out[:, :1]


# speedup vs baseline: 4.2321x; 1.0015x over previous
import functools

import jax
import jax.numpy as jnp
from jax import lax
from jax.experimental import pallas as pl
from jax.experimental.pallas import tpu as pltpu
from jax.experimental.pallas import tpu_sc as plsc

N = 10000
H = 128
E = 320000
NC = 2
NS = 16
B = 80
C = E // (NC * NS * B)
NP = 10240
RPT = NP // NS
DW = 8
HALF = NP // 2
ACCR = HALF + 8
RPC = HALF // NS

_mesh = plsc.VectorSubcoreMesh(core_axis_name="c", subcore_axis_name="s")


C2 = E // (NS * B)


def _sc_degree_body(idx_hbm, out_hbm, hist, idx_v):
    c = lax.axis_index("c")
    s = lax.axis_index("s")

    def per_array(t, _):
        def zfill(i, __):
            hist[pl.ds(i * 16, 16)] = jnp.zeros((16,), jnp.float32)
            return __
        lax.fori_loop(0, NP // 16, zfill, 0)

        pltpu.sync_copy(idx_hbm.at[2 * c + t].at[s], idx_v)
        ones16 = jnp.ones((16,), jnp.float32)

        def count(i, __):
            j = i // (B // 16)
            k = i % (B // 16)
            v = idx_v[j, pl.ds(k * 16, 16)]
            plsc.addupdate_scatter(hist, [v], ones16)
            return __
        lax.fori_loop(0, C2 * (B // 16), count, 0)
        pltpu.sync_copy(hist, out_hbm.at[2 * c + t].at[s])
        return _
    lax.fori_loop(0, 2, per_array, 0)


@functools.partial(
    pl.kernel,
    out_type=jax.ShapeDtypeStruct((4, NS, NP), jnp.float32),
    mesh=_mesh,
    scratch_types=[
        pltpu.VMEM((NP,), jnp.float32),
        pltpu.VMEM((C2, B), jnp.int32),
    ],
    compiler_params=pltpu.CompilerParams(needs_layout_passes=False),
)
def _sc_degrees(idx_hbm, out_hbm, hist, idx_v):
    _sc_degree_body(idx_hbm, out_hbm, hist, idx_v)


def _degscale_body(dh_ref, i_ref, o_ref):
    d = jnp.sum(dh_ref[...], axis=1)
    sc = lax.rsqrt(jnp.maximum(d, 1.0))
    eye = i_ref[...]
    for a in range(4):
        col = lax.dot_general(eye, sc[a:a + 1, :], (((1,), (1,)), ((), ())),
                              preferred_element_type=jnp.float32)
        o_ref[a] = jnp.broadcast_to(col, (128, DW))


def _tc_degscale(dh, eye):
    return pl.pallas_call(
        _degscale_body,
        grid=(NP // 128,),
        in_specs=[
            pl.BlockSpec((4, NS, 128), lambda i: (0, 0, i)),
            pl.BlockSpec((128, 128), lambda i: (0, 0)),
        ],
        out_specs=pl.BlockSpec((4, 128, DW), lambda i: (0, i, 0)),
        out_shape=jax.ShapeDtypeStruct((4, NP, DW), jnp.float32),
    )(dh, eye)


ZR = 8


EPT = E // (NC * NS)
BL = 128
LB = 10496


def _sc_conv_body(table_hbm, src_hbm, dst_hbm, out_hbm, idx_s, idx_d,
                  src_c, dst_c, rows, zb, gsem, ssem, acc):
    c = lax.axis_index("c")
    s = lax.axis_index("s")
    base = s * RPC

    def zr(r, _):
        def zc(k, __):
            zb[r, pl.ds(k * 16, 16)] = jnp.zeros((16,), jnp.float32)
            return __
        return lax.fori_loop(0, H // 16, zc, _)
    lax.fori_loop(0, ZR, zr, 0)

    pltpu.sync_copy(src_hbm.at[c].at[s], idx_s)
    pltpu.sync_copy(dst_hbm.at[c].at[s], idx_d)

    def pfill(i, _):
        src_c[pl.ds(i * 16, 16)] = jnp.zeros((16,), jnp.int32)
        dst_c[pl.ds(i * 16, 16)] = jnp.full((16,), HALF, jnp.int32)
        return _
    lax.fori_loop(0, LB // 16, pfill, 0)

    def cnt(i, off):
        j = i // (B // 16)
        k = i % (B // 16)
        dv = idx_d[j, pl.ds(k * 16, 16)]
        return off + jnp.sum((dv < HALF).astype(jnp.int32))
    n_lo = lax.fori_loop(0, C * (B // 16), cnt, 0)
    hb = pl.multiple_of(((n_lo + 16 + BL - 1) // BL) * BL, BL)

    def compact(i, offs):
        off_lo, off_hi = offs
        j = i // (B // 16)
        k = i % (B // 16)
        sv = idx_s[j, pl.ds(k * 16, 16)]
        dv = idx_d[j, pl.ds(k * 16, 16)]
        m_lo = dv < HALF
        plsc.store_compressed(src_c.at[pl.ds(off_lo, 16)], sv, mask=m_lo)
        plsc.store_compressed(dst_c.at[pl.ds(off_lo, 16)], dv, mask=m_lo)
        m_hi = jnp.logical_not(m_lo)
        plsc.store_compressed(src_c.at[pl.ds(off_hi, 16)], sv, mask=m_hi)
        plsc.store_compressed(dst_c.at[pl.ds(off_hi, 16)], dv - HALF,
                              mask=m_hi)
        n = jnp.sum(m_lo.astype(jnp.int32))
        return off_lo + n, off_hi + (16 - n)
    off_lo_f, off_hi_f = lax.fori_loop(0, C * (B // 16), compact, (0, hb))
    n_hi = off_hi_f - hb

    @pl.when(off_lo_f < hb)
    def _pt1():
        src_c[pl.ds(off_lo_f, 16)] = jnp.zeros((16,), jnp.int32)
        dst_c[pl.ds(off_lo_f, 16)] = jnp.full((16,), HALF, jnp.int32)

    src_c[pl.ds(off_hi_f, 16)] = jnp.zeros((16,), jnp.int32)
    dst_c[pl.ds(off_hi_f, 16)] = jnp.full((16,), HALF, jnp.int32)

    for p, nb in ((0, jnp.int32(0)), (1, hb)):
        n_p = jnp.where(p == 0, n_lo, n_hi)
        def zcp(k, __):
            pltpu.sync_copy(zb, acc.at[pl.ds(base + k * ZR, ZR)])
            return __
        lax.fori_loop(0, RPC // ZR, zcp, 0)

        @pl.when(s == 0)
        def _zt():
            pltpu.sync_copy(zb.at[pl.ds(0, 8)], acc.at[pl.ds(HALF, 8)])
        plsc.subcore_barrier()

        n_chunks = (n_p + BL - 1) // BL

        def gwin(j):
            return pl.multiple_of(nb + j * BL, BL)

        def g_cp(j, slot):
            return pltpu.make_async_copy(
                table_hbm.at[src_c.at[pl.ds(gwin(j), BL)]],
                rows.at[slot], gsem)

        def s_cp(j, slot):
            return pltpu.make_async_copy(
                rows.at[slot],
                acc.at[dst_c.at[pl.ds(gwin(j), BL)]], ssem)

        @pl.when(n_chunks > 0)
        def _pro0():
            g_cp(0, 0).start()

        def chunk(j, __):
            slot = lax.rem(j, 2)
            g_cp(j, slot).wait()
            pltpu.async_copy(rows.at[slot],
                             acc.at[dst_c.at[pl.ds(gwin(j), BL)]], ssem,
                             add=True)

            @pl.when(j >= 1)
            def _ws():
                s_cp(j - 1, lax.rem(j - 1, 2)).wait()

            @pl.when(j + 1 < n_chunks)
            def _nx():
                g_cp(j + 1, lax.rem(j + 1, 2)).start()
            return __
        lax.fori_loop(0, n_chunks, chunk, 0)

        @pl.when(n_chunks > 0)
        def _wlast():
            s_cp(n_chunks - 1, lax.rem(n_chunks - 1, 2)).wait()
        plsc.subcore_barrier()

        pltpu.sync_copy(acc.at[pl.ds(base, RPC)],
                        out_hbm.at[c].at[pl.ds(p * HALF + base, RPC)])
        plsc.subcore_barrier()


@functools.partial(
    pl.kernel,
    out_type=jax.ShapeDtypeStruct((NC, NP, H), jnp.float32),
    mesh=_mesh,
    scratch_types=[
        pltpu.VMEM((C, B), jnp.int32),
        pltpu.VMEM((C, B), jnp.int32),
        pltpu.VMEM((LB,), jnp.int32),
        pltpu.VMEM((LB,), jnp.int32),
        pltpu.VMEM((2, BL, H), jnp.float32),
        pltpu.VMEM((ZR, H), jnp.float32),
        pltpu.SemaphoreType.DMA,
        pltpu.SemaphoreType.DMA,
        pltpu.VMEM_SHARED((ACCR, H), jnp.float32),
    ],
    compiler_params=pltpu.CompilerParams(needs_layout_passes=False),
)
def _sc_conv(table_hbm, src_hbm, dst_hbm, out_hbm, idx_s, idx_d, src_c,
             dst_c, rows, zb, gsem, ssem, acc):
    _sc_conv_body(table_hbm, src_hbm, dst_hbm, out_hbm, idx_s, idx_d,
                  src_c, dst_c, rows, zb, gsem, ssem, acc)


_INV_SQRT2 = 0.7071067811865476


def _gelu(x):
    return 0.5 * x * (1.0 + lax.erf(x * _INV_SQRT2))


def _ln(x, g, b):
    mu = jnp.mean(x, axis=-1, keepdims=True)
    d = x - mu
    var = jnp.mean(d * d, axis=-1, keepdims=True)
    return d * lax.rsqrt(var + 1e-5) * g + b


def _scale_col(d_ref):
    return d_ref[:, :1]


NB = 2000


def _prep_body(x_ref, d_ref, o_ref):
    o_ref[...] = x_ref[...] * _scale_col(d_ref)


def _tc_prep(x, d_so):
    return pl.pallas_call(
        _prep_body,
        grid=(N // NB,),
        in_specs=[
            pl.BlockSpec((NB, H), lambda i: (i, 0)),
            pl.BlockSpec((NB, DW), lambda i: (i, 0)),
        ],
        out_specs=pl.BlockSpec((NB, H), lambda i: (i, 0)),
        out_shape=jax.ShapeDtypeStruct((N, H), jnp.float32),
    )(x, d_so)


def _mid_body(p_ref, dsi_ref, dso_ref, w_ref, b_ref, g_ref, be_ref, o_ref):
    agg = (p_ref[0] + p_ref[1]) * _scale_col(dsi_ref)
    z = jnp.dot(agg, w_ref[...], preferred_element_type=jnp.float32)
    y = _ln(_gelu(z + b_ref[...]), g_ref[...], be_ref[...])
    o_ref[...] = y * _scale_col(dso_ref)


def _tc_mid(p, d_si, d_so, w, b, g, be):
    vec = pl.BlockSpec((1, H), lambda i: (0, 0))
    deg = pl.BlockSpec((NB, DW), lambda i: (i, 0))
    return pl.pallas_call(
        _mid_body,
        grid=(N // NB,),
        in_specs=[
            pl.BlockSpec((NC, NB, H), lambda i: (0, i, 0)),
            deg,
            deg,
            pl.BlockSpec((H, H), lambda i: (0, 0)),
            vec, vec, vec,
        ],
        out_specs=pl.BlockSpec((NB, H), lambda i: (i, 0)),
        out_shape=jax.ShapeDtypeStruct((N, H), jnp.float32),
    )(p, d_si, d_so, w, b, g, be)


def _head_body(p_ref, dsi_ref, w2_ref, b2_ref, g_ref, be_ref, bg_ref, bb_ref,
               w3_ref, b3_ref, w4_ref, b4_ref, o_ref):
    agg = (p_ref[0] + p_ref[1]) * _scale_col(dsi_ref)
    z = jnp.dot(agg, w2_ref[...], preferred_element_type=jnp.float32)
    c2 = _ln(_gelu(z + b2_ref[...]), g_ref[...], be_ref[...])
    mu = jnp.mean(c2, axis=0, keepdims=True)
    d = c2 - mu
    var = jnp.mean(d * d, axis=0, keepdims=True)
    h = d * lax.rsqrt(var + 1e-5) * bg_ref[...] + bb_ref[...]
    h = _gelu(jnp.dot(h, w3_ref[...], preferred_element_type=jnp.float32)
              + b3_ref[...])
    o_ref[...] = (jnp.dot(h, w4_ref[...], preferred_element_type=jnp.float32)
                  + b4_ref[...])


def _tc_head(p, d_si, w2, b2, g, be, bg, bb, w3, b3, w4, b4):
    return pl.pallas_call(
        _head_body,
        out_shape=jax.ShapeDtypeStruct((N, H), jnp.float32),
    )(p, d_si, w2, b2, g, be, bg, bb, w3, b3, w4, b4)


def kernel(x_claim, x_user, edge_u2c, edge_c2u, W1_u2c, b1_u2c, W1_c2u,
           b1_c2u, W2_u2c, b2_u2c, W2_c2u, b2_c2u, ln_g, ln_b, bn_g, bn_b,
           lin1_W, lin1_b, lin2_W, lin2_b):
    e_u2c = edge_u2c.astype(jnp.int32)
    e_c2u = edge_c2u.astype(jnp.int32)

    idx4 = jnp.stack([e_c2u[0], e_c2u[1], e_u2c[0], e_u2c[1]])
    idx4 = idx4.reshape(4, NS, C2, B)
    dh = _sc_degrees(idx4)
    scales = _tc_degscale(dh, jnp.eye(128, dtype=jnp.float32))
    d_so_c = scales[0, :N]
    d_si_u = scales[1, :N]
    d_so_u = scales[2, :N]
    d_si_c = scales[3, :N]

    row = lambda v: v.reshape(1, H)

    h1 = _tc_prep(x_claim, d_so_c)
    p1 = _sc_conv(h1, e_c2u[0].reshape(NC, NS, C, B),
                  e_c2u[1].reshape(NC, NS, C, B))[:, :N]
    h2 = _tc_mid(p1, d_si_u, d_so_u, W1_c2u, row(b1_c2u), row(ln_g),
                 row(ln_b))

    p2 = _sc_conv(h2, e_u2c[0].reshape(NC, NS, C, B),
                  e_u2c[1].reshape(NC, NS, C, B))[:, :N]
    w4 = jnp.pad(lin2_W, ((0, 0), (0, H - 1)))
    b4 = jnp.pad(lin2_b, (0, H - 1)).reshape(1, H)
    out = _tc_head(p2, d_si_c, W2_u2c, row(b2_u2c), row(ln_g), row(ln_b),
                   row(bn_g), row(bn_b), lin1_W, row(lin1_b), w4, b4)
    return out[:, :1]
